# Initial kernel scaffold; baseline (speedup 1.0000x reference)
import jax
import jax.numpy as jnp
from jax.experimental import pallas as pl


def kernel(hidden_states, Wgate, Wg, Wu, Wd, Wc):
    raise NotImplementedError("write your pallas kernel here")


# speedup vs baseline: 1.7352x; 1.7352x over previous
import functools

import jax
import jax.numpy as jnp
from jax.experimental import pallas as pl

E = 16
D = 2048
D_FF = 1024
NTOK = 2048
G_O = 2


def _router_body(x_ref, wg_ref, logits_ref, wfull_ref):
    x = x_ref[...]
    logits = jax.lax.dot_general(
        x, wg_ref[...], (((1,), (1,)), ((), ())),
        preferred_element_type=jnp.float32,
    )
    logits_ref[...] = logits

    tb = logits.shape[0]
    lane = jax.lax.broadcasted_iota(jnp.int32, (tb, E), 1)
    neg = jnp.float32(-jnp.inf)

    gsum = []
    gmax = []
    garg = []
    for g in range(4):
        mask = (lane // 4) == g
        gsum.append(jnp.sum(jnp.where(mask, logits, 0.0), axis=1, keepdims=True))
        mg = jnp.max(jnp.where(mask, logits, neg), axis=1, keepdims=True)
        gmax.append(mg)
        ag = jnp.min(
            jnp.where(mask & (logits == mg), lane, 99), axis=1, keepdims=True
        ) - 4 * g
        garg.append(ag)

    eids = []
    probs = []
    for s in range(2):
        pick = gsum[2 * s + 1] > gsum[2 * s]
        e_s = jnp.where(pick, 4 * (2 * s + 1) + garg[2 * s + 1],
                        4 * (2 * s) + garg[2 * s])
        p_s = jnp.where(pick, gmax[2 * s + 1], gmax[2 * s])
        eids.append(e_s)
        probs.append(p_s)

    m = jnp.maximum(probs[0], probs[1])
    z0 = jnp.exp(probs[0] - m)
    z1 = jnp.exp(probs[1] - m)
    w0 = z0 / (z0 + z1)
    w1 = z1 / (z0 + z1)

    wfull_ref[...] = (
        jnp.where(lane == eids[0], w0, 0.0) + jnp.where(lane == eids[1], w1, 0.0)
    )


def _router(x, Wgate):
    tb = 256
    grid = (NTOK // tb,)
    return pl.pallas_call(
        _router_body,
        grid=grid,
        in_specs=[
            pl.BlockSpec((tb, D), lambda i: (i, 0)),
            pl.BlockSpec((E, D), lambda i: (0, 0)),
        ],
        out_specs=[
            pl.BlockSpec((tb, E), lambda i: (i, 0)),
            pl.BlockSpec((tb, E), lambda i: (i, 0)),
        ],
        out_shape=[
            jax.ShapeDtypeStruct((NTOK, E), jnp.float32),
            jax.ShapeDtypeStruct((NTOK, E), jnp.float32),
        ],
    )(x, Wgate)


def _ffn_body(x_ref, wgc_ref, wuc_ref, wdc_ref, wfull_ref, out_ref):
    e = pl.program_id(1)
    kf = pl.program_id(2)

    @pl.when((e % 8 == 0) & (kf == 0))
    def _():
        out_ref[...] = jnp.zeros_like(out_ref)

    x = x_ref[...]
    g = jax.lax.dot_general(x, wgc_ref[0], (((1,), (1,)), ((), ())),
                            preferred_element_type=jnp.float32)
    u = jax.lax.dot_general(x, wuc_ref[0], (((1,), (1,)), ((), ())),
                            preferred_element_type=jnp.float32)
    hc = (g * jax.nn.sigmoid(g)) * u
    z = jax.lax.dot_general(hc, wdc_ref[0], (((1,), (1,)), ((), ())),
                            preferred_element_type=jnp.float32)

    lane = jax.lax.broadcasted_iota(jnp.int32, wfull_ref.shape, 1)
    wcol = jnp.sum(jnp.where(lane == e, wfull_ref[...], 0.0), axis=1,
                   keepdims=True)
    out_ref[...] += wcol * z


def _dense_ffn(x, Wg, Wu, Wd, w_full):
    tb = 1024
    ffc = 512
    grid = (NTOK // tb, E, D_FF // ffc)
    cs = D // G_O
    return pl.pallas_call(
        _ffn_body,
        grid=grid,
        in_specs=[
            pl.BlockSpec((tb, D), lambda t, e, k: (t, 0)),
            pl.BlockSpec((1, ffc, D), lambda t, e, k: (e, k, 0)),
            pl.BlockSpec((1, ffc, D), lambda t, e, k: (e, k, 0)),
            pl.BlockSpec((1, cs, ffc), lambda t, e, k: (e, 0, k)),
            pl.BlockSpec((tb, E), lambda t, e, k: (t, 0)),
        ],
        out_specs=pl.BlockSpec((tb, cs), lambda t, e, k: (t, e // 8)),
        out_shape=jax.ShapeDtypeStruct((NTOK, D), jnp.float32),
    )(x, Wg, Wu, Wd, w_full)


def _wc_body(mid_ref, wc_ref, out_ref):
    out_ref[...] = jax.lax.dot_general(
        mid_ref[...], wc_ref[...], (((1,), (1,)), ((), ())),
        preferred_element_type=jnp.float32,
    )


def _wc_matmul(mid, Wc):
    tb = 256
    return pl.pallas_call(
        _wc_body,
        grid=(NTOK // tb,),
        in_specs=[
            pl.BlockSpec((tb, D), lambda i: (i, 0)),
            pl.BlockSpec((D, D), lambda i: (0, 0)),
        ],
        out_specs=pl.BlockSpec((tb, D), lambda i: (i, 0)),
        out_shape=jax.ShapeDtypeStruct((NTOK, D), jnp.float32),
    )(mid, Wc)


def kernel(hidden_states, Wgate, Wg, Wu, Wd, Wc):
    b, s_len, d = hidden_states.shape
    x = hidden_states.reshape(b * s_len, d)
    logits, w_full = _router(x, Wgate)
    mid = _dense_ffn(x, Wg, Wu, Wd, w_full)
    out = _wc_matmul(mid, Wc)
    return out.reshape(b, s_len, d), logits
